# Initial kernel scaffold; baseline (speedup 1.0000x reference)
import jax
import jax.numpy as jnp
from jax.experimental import pallas as pl


def kernel(x, y, centers):
    raise NotImplementedError("write your pallas kernel here")


# speedup vs baseline: 43.2295x; 43.2295x over previous
import functools

import jax
import jax.numpy as jnp
from jax import lax
from jax.experimental import pallas as pl
from jax.experimental.pallas import tpu as pltpu

B = 10000
D = 32
K = 512
NCLS = 10
RB = 2000
G = B // RB
CPAD = 128


def _body(x_ref, y_ref, c_ref, loss_ref, acc_ref, conf_ref, loss_acc):
    i = pl.program_id(0)

    @pl.when(i == 0)
    def _init():
        conf_ref[...] = jnp.zeros_like(conf_ref)
        loss_acc[0, 0] = 0.0

    xb = x_ref[...]
    c = c_ref[...]
    xn = jnp.sum(xb * xb, axis=1, keepdims=True)
    cn = jnp.sum(c * c, axis=1)
    dot = lax.dot_general(xb, c, (((1,), (1,)), ((), ())),
                          preferred_element_type=jnp.float32,
                          precision=lax.Precision.HIGHEST)
    dist = xn - 2.0 * dot + cn[None, :]
    minv = jnp.min(dist, axis=1, keepdims=True)
    kidx = lax.broadcasted_iota(jnp.int32, (RB, K), 1)
    y_p = jnp.min(jnp.where(dist == minv, kidx, K), axis=1)

    loss_acc[0, 0] += jnp.sum(minv)

    yb = y_ref[0, 0, :]
    weight = (kidx == y_p[:, None]).astype(jnp.float32)
    oh_y = (lax.broadcasted_iota(jnp.int32, (RB, CPAD), 1)
            == yb[:, None]).astype(jnp.float32)
    conf_ref[...] += lax.dot_general(
        weight, oh_y, (((0,), (0,)), ((), ())),
        preferred_element_type=jnp.float32)

    @pl.when(i == G - 1)
    def _fini():
        correct = jnp.sum(jnp.max(conf_ref[...], axis=1))
        acc_ref[...] = jnp.reshape(correct * (1.0 / B), (1, 1))
        loss_ref[...] = jnp.reshape(loss_acc[0, 0], (1, 1))


@jax.jit
def kernel(x, y, centers):
    y3 = y.astype(jnp.int32).reshape(G, 1, RB)
    loss, acc = pl.pallas_call(
        _body,
        grid=(G,),
        in_specs=[
            pl.BlockSpec((RB, D), lambda i: (i, 0)),
            pl.BlockSpec((1, 1, RB), lambda i: (i, 0, 0)),
            pl.BlockSpec((K, D), lambda i: (0, 0)),
        ],
        out_specs=[
            pl.BlockSpec((1, 1), lambda i: (0, 0)),
            pl.BlockSpec((1, 1), lambda i: (0, 0)),
        ],
        out_shape=[
            jax.ShapeDtypeStruct((1, 1), jnp.float32),
            jax.ShapeDtypeStruct((1, 1), jnp.float32),
        ],
        scratch_shapes=[
            pltpu.VMEM((K, CPAD), jnp.float32),
            pltpu.SMEM((1, 1), jnp.float32),
        ],
    )(x, y3, centers)
    return (loss.reshape(()), acc.reshape(()))
